# baseline (speedup 1.0000x reference)
import functools

import jax
import jax.numpy as jnp
from jax import lax
from jax.experimental import pallas as pl
from jax.experimental.pallas import tpu as pltpu
from jax.experimental.pallas import tpu_sc as plsc

B, S, D, V = 4096, 200, 64, 1000000
NC, NS = 2, 16
NW = NC * NS
ROWS = B * S
RPW = ROWS // NW
CH = 128
NCH = RPW // CH
NBUF = 6
PD = 3
POSREP = 320


def _body(x_ref, tok_ref, pos_ref, out_ref, idx_all, pos2, bufs, gsem, osem):
    wid = lax.axis_index("s") * NC + lax.axis_index("c")
    irow0 = wid * NCH
    orow0 = wid * RPW

    pltpu.sync_copy(x_ref.at[pl.ds(irow0, NCH)], idx_all)
    pltpu.sync_copy(pos_ref, pos2.at[pl.ds(0, S)])
    pltpu.sync_copy(pos_ref.at[pl.ds(0, POSREP - S)], pos2.at[pl.ds(S, POSREP - S)])

    def start_gather(g):
        slot = lax.rem(g, NBUF)
        pltpu.async_copy(tok_ref.at[idx_all.at[g]], bufs.at[slot], gsem.at[slot])

    def wait_gather(slot):
        pltpu.make_async_copy(
            tok_ref.at[idx_all.at[0]], bufs.at[slot], gsem.at[slot]).wait()

    def start_out(g, slot):
        pltpu.async_copy(
            bufs.at[slot], out_ref.at[pl.ds(orow0 + g * CH, CH)], osem.at[slot])

    def wait_out(slot):
        pltpu.make_async_copy(
            bufs.at[slot], out_ref.at[pl.ds(orow0, CH)], osem.at[slot]).wait()

    for g in range(PD):
        start_gather(g)

    def chunk(lg, carry):
        slot = lax.rem(lg, NBUF)
        gn = lg + PD

        @pl.when(gn < NCH)
        def _():
            @pl.when(lg >= PD)
            def _():
                wait_out(lax.rem(gn, NBUF))
            start_gather(gn)

        wait_gather(slot)

        o = lax.rem(lg * CH, S)

        @plsc.parallel_loop(0, CH, step=1, unroll=8)
        def _add(r):
            for q in range(D // 16):
                bufs[slot, r, pl.ds(q * 16, 16)] = (
                    bufs[slot, r, pl.ds(q * 16, 16)]
                    + pos2[o + r, pl.ds(q * 16, 16)])

        start_out(lg, slot)
        return carry

    lax.fori_loop(0, NCH, chunk, 0)
    for k in range(NBUF):
        wait_out(k)


_sc_call = functools.partial(
    pl.kernel,
    out_type=jax.ShapeDtypeStruct((ROWS, D), jnp.float32),
    mesh=plsc.VectorSubcoreMesh(
        core_axis_name="c", subcore_axis_name="s",
        num_cores=NC, num_subcores=NS),
    scratch_types=[
        pltpu.VMEM((NCH, CH), jnp.int32),
        pltpu.VMEM((POSREP, D), jnp.float32),
        pltpu.VMEM((NBUF, CH, D), jnp.float32),
        pltpu.SemaphoreType.DMA((NBUF,)),
        pltpu.SemaphoreType.DMA((NBUF,)),
    ],
    compiler_params=pltpu.CompilerParams(use_tc_tiling_on_sc=False),
)(_body)


def kernel(x, token_table, position_table):
    x2 = x.astype(jnp.int32).reshape(ROWS // CH, CH)
    out = _sc_call(x2, token_table, position_table)
    return out.reshape(B, S, D)
